# baseline (speedup 1.0000x reference)
import jax
import jax.numpy as jnp
from jax.experimental import pallas as pl
from jax.experimental.pallas import tpu as pltpu

N = 4096
D = 512
BM = 128
NB = N // BM


def _feat(vb, wslat_ref, bsla_ref):
    s = jnp.tanh(jnp.dot(vb, wslat_ref[...],
                         preferred_element_type=jnp.float32) + bsla_ref[...])
    return jnp.sum(s, axis=0, keepdims=True)


def _copy(meta_ref, mb_scr, sem, k, slot):
    return pltpu.make_async_copy(
        meta_ref.at[pl.ds(k * BM, BM), :], mb_scr.at[slot], sem.at[slot])


def _pair_body(emb_ref, w1t_ref, w2t_ref, bfc1_ref, bfc2_ref, bias1_ref,
               bias2_ref, p1_ref, p2_ref, wslat_ref, bsla_ref, asla_ref,
               meta1_ref, meta2_ref, out_ref,
               mb1_scr, mb2_scr, h1_scr, h2_scr, v1_scr, v2_scr,
               acc1_scr, acc2_scr, sem1, sem2):
    i = pl.program_id(0)

    @pl.when(i == 0)
    def _init():
        _copy(meta1_ref, mb1_scr, sem1, 0, 0).start()
        _copy(meta2_ref, mb2_scr, sem2, 0, 0).start()
        h1 = jnp.dot(emb_ref[...], w1t_ref[...],
                     preferred_element_type=jnp.float32) + bfc1_ref[...]
        h1_scr[...] = h1.astype(jnp.bfloat16)
        h2 = jnp.dot(emb_ref[...], w2t_ref[...],
                     preferred_element_type=jnp.float32) + bfc2_ref[...]
        h2_scr[...] = h2.astype(jnp.bfloat16)
        acc1_scr[...] = jnp.zeros_like(acc1_scr)
        acc2_scr[...] = jnp.zeros_like(acc2_scr)

    @pl.when(i < NB)
    def _heavy():
        @pl.when(i + 1 < NB)
        def _issue_next():
            nxt = i + 1
            nslot = jax.lax.rem(nxt, 2)
            _copy(meta1_ref, mb1_scr, sem1, nxt, nslot).start()
            _copy(meta2_ref, mb2_scr, sem2, nxt, nslot).start()

        slot = jax.lax.rem(i, 2)
        _copy(meta1_ref, mb1_scr, sem1, i, slot).wait()
        _copy(meta2_ref, mb2_scr, sem2, i, slot).wait()

        m1 = mb1_scr[slot].astype(jnp.bfloat16)
        out1 = jnp.dot(m1, h1_scr[...],
                       preferred_element_type=jnp.float32) + bias1_ref[...]
        vb1 = jnp.where(out1 >= 0, out1,
                        p1_ref[0, 0] * out1).astype(jnp.bfloat16)
        v1_scr[pl.ds(i * BM, BM), :] = vb1
        m2 = mb2_scr[slot].astype(jnp.bfloat16)
        out2 = jnp.dot(m2, h2_scr[...],
                       preferred_element_type=jnp.float32) + bias2_ref[...]
        vb2 = jnp.where(out2 >= 0, out2,
                        p2_ref[0, 0] * out2).astype(jnp.bfloat16)
        v2_scr[pl.ds(i * BM, BM), :] = vb2

        acc1_scr[...] += _feat(vb1, wslat_ref, bsla_ref)
        acc2_scr[...] += _feat(vb2, wslat_ref, bsla_ref)

    @pl.when(i >= NB)
    def _combine():
        j = i - NB
        la = jnp.sum(asla_ref[...] * acc1_scr[...] * (1.0 / N),
                     axis=1, keepdims=True)
        lb = jnp.sum(asla_ref[...] * acc2_scr[...] * (1.0 / N),
                     axis=1, keepdims=True)
        m = jnp.maximum(la, lb)
        ea = jnp.exp(la - m)
        eb = jnp.exp(lb - m)
        inv = 1.0 / (ea + eb)
        b1 = ea * inv
        b2 = eb * inv
        v1 = v1_scr[pl.ds(j * BM, BM), :].astype(jnp.float32)
        v2 = v2_scr[pl.ds(j * BM, BM), :].astype(jnp.float32)
        out_ref[...] = v1 * b1 + v2 * b2


def _pair_call(emb_bf, w1t, w2t, bfc1, bfc2, bias1, bias2, p1, p2,
               wslat, bsla, asla, meta1, meta2):
    const = lambda i: (0, 0)
    anyspec = pl.BlockSpec(memory_space=pl.ANY)
    return pl.pallas_call(
        _pair_body,
        grid=(2 * NB,),
        in_specs=[
            pl.BlockSpec((N, D), const),
            pl.BlockSpec((D, D), const),
            pl.BlockSpec((D, D), const),
            pl.BlockSpec((1, D), const),
            pl.BlockSpec((1, D), const),
            pl.BlockSpec((1, D), const),
            pl.BlockSpec((1, D), const),
            pl.BlockSpec((1, 1), const),
            pl.BlockSpec((1, 1), const),
            pl.BlockSpec((D, D), const),
            pl.BlockSpec((1, D), const),
            pl.BlockSpec((1, D), const),
            anyspec,
            anyspec,
        ],
        out_specs=pl.BlockSpec((BM, D),
                               lambda i: (jnp.clip(i - NB, 0, NB - 1), 0)),
        out_shape=jax.ShapeDtypeStruct((N, D), jnp.float32),
        scratch_shapes=[
            pltpu.VMEM((2, BM, N), jnp.float32),
            pltpu.VMEM((2, BM, N), jnp.float32),
            pltpu.VMEM((N, D), jnp.bfloat16),
            pltpu.VMEM((N, D), jnp.bfloat16),
            pltpu.VMEM((N, D), jnp.bfloat16),
            pltpu.VMEM((N, D), jnp.bfloat16),
            pltpu.VMEM((1, D), jnp.float32),
            pltpu.VMEM((1, D), jnp.float32),
            pltpu.SemaphoreType.DMA((2,)),
            pltpu.SemaphoreType.DMA((2,)),
        ],
    )(emb_bf, w1t, w2t, bfc1, bfc2, bias1, bias2, p1, p2, wslat, bsla, asla,
      meta1, meta2)


@jax.jit
def kernel(emb_mi, emb_di, meta_mdm, meta_mdmdm, meta_dmd, meta_dmdmd,
           W_mdm, bfc_mdm, bias_mdm, p_mdm,
           W_mdmdm, bfc_mdmdm, bias_mdmdm, p_mdmdm,
           W_dmd, bfc_dmd, bias_dmd, p_dmd,
           W_dmdmd, bfc_dmdmd, bias_dmdmd, p_dmdmd,
           W_sla, b_sla, a_sla):
    wslat = W_sla.T.astype(jnp.bfloat16)
    bsla = b_sla.reshape(1, D)
    asla = a_sla.reshape(1, D)

    out_mi = _pair_call(
        emb_mi.astype(jnp.bfloat16),
        W_mdm.T.astype(jnp.bfloat16), W_mdmdm.T.astype(jnp.bfloat16),
        bfc_mdm.reshape(1, D), bfc_mdmdm.reshape(1, D),
        bias_mdm.reshape(1, D), bias_mdmdm.reshape(1, D),
        p_mdm.reshape(1, 1), p_mdmdm.reshape(1, 1),
        wslat, bsla, asla, meta_mdm, meta_mdmdm)
    out_di = _pair_call(
        emb_di.astype(jnp.bfloat16),
        W_dmd.T.astype(jnp.bfloat16), W_dmdmd.T.astype(jnp.bfloat16),
        bfc_dmd.reshape(1, D), bfc_dmdmd.reshape(1, D),
        bias_dmd.reshape(1, D), bias_dmdmd.reshape(1, D),
        p_dmd.reshape(1, 1), p_dmdmd.reshape(1, 1),
        wslat, bsla, asla, meta_dmd, meta_dmdmd)
    return out_mi, out_di


# speedup vs baseline: 1.1775x; 1.1775x over previous
import jax
import jax.numpy as jnp
from jax.experimental import pallas as pl
from jax.experimental.pallas import tpu as pltpu

N = 4096
D = 512
BM = 256
NB = N // BM


def _feat(vb, wslat_ref, bsla_ref):
    s = jnp.tanh(jnp.dot(vb, wslat_ref[...],
                         preferred_element_type=jnp.float32) + bsla_ref[...])
    return jnp.sum(s, axis=0, keepdims=True)


def _copy(meta_ref, mb_scr, sem, k, slot):
    return pltpu.make_async_copy(
        meta_ref.at[pl.ds(k * BM, BM), :], mb_scr.at[slot], sem.at[slot])


def _pair_body(emb_ref, w1t_ref, w2t_ref, bfc1_ref, bfc2_ref, bias1_ref,
               bias2_ref, p1_ref, p2_ref, wslat_ref, bsla_ref, asla_ref,
               meta1_ref, meta2_ref, out_ref,
               mb1_scr, mb2_scr, h1_scr, h2_scr, v1_scr, v2_scr,
               acc1_scr, acc2_scr, sem1, sem2):
    i = pl.program_id(0)

    @pl.when(i == 0)
    def _init():
        _copy(meta1_ref, mb1_scr, sem1, 0, 0).start()
        _copy(meta2_ref, mb2_scr, sem2, 0, 0).start()
        h1 = jnp.dot(emb_ref[...], w1t_ref[...],
                     preferred_element_type=jnp.float32) + bfc1_ref[...]
        h1_scr[...] = h1.astype(jnp.bfloat16)
        h2 = jnp.dot(emb_ref[...], w2t_ref[...],
                     preferred_element_type=jnp.float32) + bfc2_ref[...]
        h2_scr[...] = h2.astype(jnp.bfloat16)
        acc1_scr[...] = jnp.zeros_like(acc1_scr)
        acc2_scr[...] = jnp.zeros_like(acc2_scr)

    @pl.when(i == 0)
    def _init2():
        _copy(meta1_ref, mb1_scr, sem1, 1, 1).start()
        _copy(meta2_ref, mb2_scr, sem2, 1, 1).start()

    @pl.when(i < NB)
    def _heavy():
        @pl.when(i + 2 < NB)
        def _issue_next():
            nxt = i + 2
            nslot = jax.lax.rem(nxt, 3)
            _copy(meta1_ref, mb1_scr, sem1, nxt, nslot).start()
            _copy(meta2_ref, mb2_scr, sem2, nxt, nslot).start()

        slot = jax.lax.rem(i, 3)
        _copy(meta1_ref, mb1_scr, sem1, i, slot).wait()
        _copy(meta2_ref, mb2_scr, sem2, i, slot).wait()

        m1 = mb1_scr[slot].astype(jnp.bfloat16)
        out1 = jnp.dot(m1, h1_scr[...],
                       preferred_element_type=jnp.float32) + bias1_ref[...]
        vb1 = jnp.where(out1 >= 0, out1,
                        p1_ref[0, 0] * out1).astype(jnp.bfloat16)
        v1_scr[pl.ds(i * BM, BM), :] = vb1
        m2 = mb2_scr[slot].astype(jnp.bfloat16)
        out2 = jnp.dot(m2, h2_scr[...],
                       preferred_element_type=jnp.float32) + bias2_ref[...]
        vb2 = jnp.where(out2 >= 0, out2,
                        p2_ref[0, 0] * out2).astype(jnp.bfloat16)
        v2_scr[pl.ds(i * BM, BM), :] = vb2

        acc1_scr[...] += _feat(vb1, wslat_ref, bsla_ref)
        acc2_scr[...] += _feat(vb2, wslat_ref, bsla_ref)

    @pl.when(i >= NB)
    def _combine():
        j = i - NB
        la = jnp.sum(asla_ref[...] * acc1_scr[...] * (1.0 / N),
                     axis=1, keepdims=True)
        lb = jnp.sum(asla_ref[...] * acc2_scr[...] * (1.0 / N),
                     axis=1, keepdims=True)
        m = jnp.maximum(la, lb)
        ea = jnp.exp(la - m)
        eb = jnp.exp(lb - m)
        inv = 1.0 / (ea + eb)
        b1 = ea * inv
        b2 = eb * inv
        v1 = v1_scr[pl.ds(j * BM, BM), :].astype(jnp.float32)
        v2 = v2_scr[pl.ds(j * BM, BM), :].astype(jnp.float32)
        out_ref[...] = v1 * b1 + v2 * b2


def _pair_call(emb_bf, w1t, w2t, bfc1, bfc2, bias1, bias2, p1, p2,
               wslat, bsla, asla, meta1, meta2):
    const = lambda i: (0, 0)
    anyspec = pl.BlockSpec(memory_space=pl.ANY)
    return pl.pallas_call(
        _pair_body,
        grid=(2 * NB,),
        in_specs=[
            pl.BlockSpec((N, D), const),
            pl.BlockSpec((D, D), const),
            pl.BlockSpec((D, D), const),
            pl.BlockSpec((1, D), const),
            pl.BlockSpec((1, D), const),
            pl.BlockSpec((1, D), const),
            pl.BlockSpec((1, D), const),
            pl.BlockSpec((1, 1), const),
            pl.BlockSpec((1, 1), const),
            pl.BlockSpec((D, D), const),
            pl.BlockSpec((1, D), const),
            pl.BlockSpec((1, D), const),
            anyspec,
            anyspec,
        ],
        out_specs=pl.BlockSpec((BM, D),
                               lambda i: (jnp.clip(i - NB, 0, NB - 1), 0)),
        out_shape=jax.ShapeDtypeStruct((N, D), jnp.float32),
        scratch_shapes=[
            pltpu.VMEM((3, BM, N), jnp.float32),
            pltpu.VMEM((3, BM, N), jnp.float32),
            pltpu.VMEM((N, D), jnp.bfloat16),
            pltpu.VMEM((N, D), jnp.bfloat16),
            pltpu.VMEM((N, D), jnp.bfloat16),
            pltpu.VMEM((N, D), jnp.bfloat16),
            pltpu.VMEM((1, D), jnp.float32),
            pltpu.VMEM((1, D), jnp.float32),
            pltpu.SemaphoreType.DMA((3,)),
            pltpu.SemaphoreType.DMA((3,)),
        ],
    )(emb_bf, w1t, w2t, bfc1, bfc2, bias1, bias2, p1, p2, wslat, bsla, asla,
      meta1, meta2)


@jax.jit
def kernel(emb_mi, emb_di, meta_mdm, meta_mdmdm, meta_dmd, meta_dmdmd,
           W_mdm, bfc_mdm, bias_mdm, p_mdm,
           W_mdmdm, bfc_mdmdm, bias_mdmdm, p_mdmdm,
           W_dmd, bfc_dmd, bias_dmd, p_dmd,
           W_dmdmd, bfc_dmdmd, bias_dmdmd, p_dmdmd,
           W_sla, b_sla, a_sla):
    wslat = W_sla.T.astype(jnp.bfloat16)
    bsla = b_sla.reshape(1, D)
    asla = a_sla.reshape(1, D)

    out_mi = _pair_call(
        emb_mi.astype(jnp.bfloat16),
        W_mdm.T.astype(jnp.bfloat16), W_mdmdm.T.astype(jnp.bfloat16),
        bfc_mdm.reshape(1, D), bfc_mdmdm.reshape(1, D),
        bias_mdm.reshape(1, D), bias_mdmdm.reshape(1, D),
        p_mdm.reshape(1, 1), p_mdmdm.reshape(1, 1),
        wslat, bsla, asla, meta_mdm, meta_mdmdm)
    out_di = _pair_call(
        emb_di.astype(jnp.bfloat16),
        W_dmd.T.astype(jnp.bfloat16), W_dmdmd.T.astype(jnp.bfloat16),
        bfc_dmd.reshape(1, D), bfc_dmdmd.reshape(1, D),
        bias_dmd.reshape(1, D), bias_dmdmd.reshape(1, D),
        p_dmd.reshape(1, 1), p_dmdmd.reshape(1, 1),
        wslat, bsla, asla, meta_dmd, meta_dmdmd)
    return out_mi, out_di


# speedup vs baseline: 1.2353x; 1.0490x over previous
import jax
import jax.numpy as jnp
from jax.experimental import pallas as pl
from jax.experimental.pallas import tpu as pltpu

N = 4096
D = 512
BM = 256
NB = N // BM


def _feat(vb, wslat_ref, bsla_ref):
    s = jnp.tanh(jnp.dot(vb, wslat_ref[...],
                         preferred_element_type=jnp.float32) + bsla_ref[...])
    return jnp.sum(s, axis=0, keepdims=True)


def _copy(meta_ref, mb_scr, sem, k, slot):
    return pltpu.make_async_copy(
        meta_ref.at[pl.ds(k * BM, BM), :], mb_scr.at[slot], sem.at[slot])


def _h(emb_ref, wt_ref, bfc_ref):
    h = jnp.dot(emb_ref[...], wt_ref[...],
                preferred_element_type=jnp.float32) + bfc_ref[...]
    return h.astype(jnp.bfloat16)


def _spmm(mb_scr, slot, h_scr, bias_ref, p_ref):
    out = jnp.dot(mb_scr[slot].astype(jnp.bfloat16), h_scr[...],
                  preferred_element_type=jnp.float32) + bias_ref[...]
    return jnp.where(out >= 0, out, p_ref[0, 0] * out).astype(jnp.bfloat16)


def _betas(asla_ref, acca_scr, accb_scr):
    la = jnp.sum(asla_ref[...] * acca_scr[...] * (1.0 / N),
                 axis=1, keepdims=True)
    lb = jnp.sum(asla_ref[...] * accb_scr[...] * (1.0 / N),
                 axis=1, keepdims=True)
    m = jnp.maximum(la, lb)
    ea = jnp.exp(la - m)
    eb = jnp.exp(lb - m)
    inv = 1.0 / (ea + eb)
    return ea * inv, eb * inv


def _body(embmi_ref, embdi_ref, w1t_ref, w2t_ref, w3t_ref, w4t_ref,
          bfc1_ref, bfc2_ref, bfc3_ref, bfc4_ref,
          bias1_ref, bias2_ref, bias3_ref, bias4_ref,
          p1_ref, p2_ref, p3_ref, p4_ref, wslat_ref, bsla_ref, asla_ref,
          meta1_ref, meta2_ref, meta3_ref, meta4_ref,
          outmi_ref, outdi_ref,
          mba_scr, mbb_scr, h1_scr, h2_scr, v1_scr, v2_scr, v3_scr, v4_scr,
          acc1_scr, acc2_scr, acc3_scr, acc4_scr, sema, semb):
    i = pl.program_id(0)

    @pl.when(i == 0)
    def _init1():
        _copy(meta1_ref, mba_scr, sema, 0, 0).start()
        _copy(meta2_ref, mbb_scr, semb, 0, 0).start()
        h1_scr[...] = _h(embmi_ref, w1t_ref, bfc1_ref)
        h2_scr[...] = _h(embmi_ref, w2t_ref, bfc2_ref)
        acc1_scr[...] = jnp.zeros_like(acc1_scr)
        acc2_scr[...] = jnp.zeros_like(acc2_scr)

    @pl.when(i < NB)
    def _heavy1():
        @pl.when(i + 1 < NB)
        def _issue_same():
            nxt = i + 1
            nslot = jax.lax.rem(nxt, 2)
            _copy(meta1_ref, mba_scr, sema, nxt, nslot).start()
            _copy(meta2_ref, mbb_scr, semb, nxt, nslot).start()

        @pl.when(i + 1 == NB)
        def _issue_next_pair():
            nslot = jax.lax.rem(NB, 2)
            _copy(meta3_ref, mba_scr, sema, 0, nslot).start()
            _copy(meta4_ref, mbb_scr, semb, 0, nslot).start()

        slot = jax.lax.rem(i, 2)
        _copy(meta1_ref, mba_scr, sema, i, slot).wait()
        _copy(meta2_ref, mbb_scr, semb, i, slot).wait()

        vb1 = _spmm(mba_scr, slot, h1_scr, bias1_ref, p1_ref)
        v1_scr[pl.ds(i * BM, BM), :] = vb1
        vb2 = _spmm(mbb_scr, slot, h2_scr, bias2_ref, p2_ref)
        v2_scr[pl.ds(i * BM, BM), :] = vb2
        acc1_scr[...] += _feat(vb1, wslat_ref, bsla_ref)
        acc2_scr[...] += _feat(vb2, wslat_ref, bsla_ref)

    @pl.when(i == NB)
    def _init2():
        h1_scr[...] = _h(embdi_ref, w3t_ref, bfc3_ref)
        h2_scr[...] = _h(embdi_ref, w4t_ref, bfc4_ref)
        acc3_scr[...] = jnp.zeros_like(acc3_scr)
        acc4_scr[...] = jnp.zeros_like(acc4_scr)

    @pl.when(jnp.logical_and(i >= NB, i < 2 * NB))
    def _heavy2():
        j = i - NB

        @pl.when(j + 1 < NB)
        def _issue_same2():
            nxt = j + 1
            nslot = jax.lax.rem(NB + nxt, 2)
            _copy(meta3_ref, mba_scr, sema, nxt, nslot).start()
            _copy(meta4_ref, mbb_scr, semb, nxt, nslot).start()

        slot = jax.lax.rem(NB + j, 2)
        _copy(meta3_ref, mba_scr, sema, j, slot).wait()
        _copy(meta4_ref, mbb_scr, semb, j, slot).wait()

        vb3 = _spmm(mba_scr, slot, h1_scr, bias3_ref, p3_ref)
        v3_scr[pl.ds(j * BM, BM), :] = vb3
        vb4 = _spmm(mbb_scr, slot, h2_scr, bias4_ref, p4_ref)
        v4_scr[pl.ds(j * BM, BM), :] = vb4
        acc3_scr[...] += _feat(vb3, wslat_ref, bsla_ref)
        acc4_scr[...] += _feat(vb4, wslat_ref, bsla_ref)

        b1, b2 = _betas(asla_ref, acc1_scr, acc2_scr)
        v1 = v1_scr[pl.ds(j * BM, BM), :].astype(jnp.float32)
        v2 = v2_scr[pl.ds(j * BM, BM), :].astype(jnp.float32)
        outmi_ref[...] = v1 * b1 + v2 * b2

    @pl.when(i >= 2 * NB)
    def _combine2():
        j = i - 2 * NB
        b3, b4 = _betas(asla_ref, acc3_scr, acc4_scr)
        v3 = v3_scr[pl.ds(j * BM, BM), :].astype(jnp.float32)
        v4 = v4_scr[pl.ds(j * BM, BM), :].astype(jnp.float32)
        outdi_ref[...] = v3 * b3 + v4 * b4


@jax.jit
def kernel(emb_mi, emb_di, meta_mdm, meta_mdmdm, meta_dmd, meta_dmdmd,
           W_mdm, bfc_mdm, bias_mdm, p_mdm,
           W_mdmdm, bfc_mdmdm, bias_mdmdm, p_mdmdm,
           W_dmd, bfc_dmd, bias_dmd, p_dmd,
           W_dmdmd, bfc_dmdmd, bias_dmdmd, p_dmdmd,
           W_sla, b_sla, a_sla):
    const = lambda i: (0, 0)
    anyspec = pl.BlockSpec(memory_space=pl.ANY)
    outmi_spec = pl.BlockSpec((BM, D),
                              lambda i: (jnp.clip(i - NB, 0, NB - 1), 0))
    outdi_spec = pl.BlockSpec((BM, D),
                              lambda i: (jnp.clip(i - 2 * NB, 0, NB - 1), 0))
    out_mi, out_di = pl.pallas_call(
        _body,
        grid=(3 * NB,),
        in_specs=[
            pl.BlockSpec((N, D), const),
            pl.BlockSpec((N, D), const),
            pl.BlockSpec((D, D), const),
            pl.BlockSpec((D, D), const),
            pl.BlockSpec((D, D), const),
            pl.BlockSpec((D, D), const),
            pl.BlockSpec((1, D), const),
            pl.BlockSpec((1, D), const),
            pl.BlockSpec((1, D), const),
            pl.BlockSpec((1, D), const),
            pl.BlockSpec((1, D), const),
            pl.BlockSpec((1, D), const),
            pl.BlockSpec((1, D), const),
            pl.BlockSpec((1, D), const),
            pl.BlockSpec((1, 1), const),
            pl.BlockSpec((1, 1), const),
            pl.BlockSpec((1, 1), const),
            pl.BlockSpec((1, 1), const),
            pl.BlockSpec((D, D), const),
            pl.BlockSpec((1, D), const),
            pl.BlockSpec((1, D), const),
            anyspec,
            anyspec,
            anyspec,
            anyspec,
        ],
        out_specs=[outmi_spec, outdi_spec],
        out_shape=[
            jax.ShapeDtypeStruct((N, D), jnp.float32),
            jax.ShapeDtypeStruct((N, D), jnp.float32),
        ],
        scratch_shapes=[
            pltpu.VMEM((2, BM, N), jnp.float32),
            pltpu.VMEM((2, BM, N), jnp.float32),
            pltpu.VMEM((N, D), jnp.bfloat16),
            pltpu.VMEM((N, D), jnp.bfloat16),
            pltpu.VMEM((N, D), jnp.bfloat16),
            pltpu.VMEM((N, D), jnp.bfloat16),
            pltpu.VMEM((N, D), jnp.bfloat16),
            pltpu.VMEM((N, D), jnp.bfloat16),
            pltpu.VMEM((1, D), jnp.float32),
            pltpu.VMEM((1, D), jnp.float32),
            pltpu.VMEM((1, D), jnp.float32),
            pltpu.VMEM((1, D), jnp.float32),
            pltpu.SemaphoreType.DMA((2,)),
            pltpu.SemaphoreType.DMA((2,)),
        ],
    )(emb_mi.astype(jnp.bfloat16), emb_di.astype(jnp.bfloat16),
      W_mdm.T.astype(jnp.bfloat16), W_mdmdm.T.astype(jnp.bfloat16),
      W_dmd.T.astype(jnp.bfloat16), W_dmdmd.T.astype(jnp.bfloat16),
      bfc_mdm.reshape(1, D), bfc_mdmdm.reshape(1, D),
      bfc_dmd.reshape(1, D), bfc_dmdmd.reshape(1, D),
      bias_mdm.reshape(1, D), bias_mdmdm.reshape(1, D),
      bias_dmd.reshape(1, D), bias_dmdmd.reshape(1, D),
      p_mdm.reshape(1, 1), p_mdmdm.reshape(1, 1),
      p_dmd.reshape(1, 1), p_dmdmd.reshape(1, 1),
      W_sla.T.astype(jnp.bfloat16), b_sla.reshape(1, D), a_sla.reshape(1, D),
      meta_mdm, meta_mdmdm, meta_dmd, meta_dmdmd)
    return out_mi, out_di
